# baseline (speedup 1.0000x reference)
import functools

import jax
import jax.numpy as jnp
from jax.experimental import pallas as pl
from jax.experimental.pallas import tpu as pltpu


def _round_up(n, m):
    return ((n + m - 1) // m) * m


def _se_kernel(x_ref, w1_ref, b1_ref, w2_ref, b2_ref, o_ref, *, inv_hw):
    x = x_ref[...]
    pooled = jnp.sum(x, axis=2, dtype=jnp.float32) * inv_hw
    h = jnp.dot(pooled, w1_ref[...],
                preferred_element_type=jnp.float32)
    h = jnp.maximum(h + b1_ref[...], 0.0)
    s = jnp.dot(h, w2_ref[...],
                preferred_element_type=jnp.float32)
    s = jax.nn.sigmoid(s + b2_ref[...])
    o_ref[...] = (x * s[:, :, None].astype(x.dtype)).astype(o_ref.dtype)


def kernel(x_nchw, w1, b1, w2, b2):
    B, C, H, W = x_nchw.shape
    Cr = w1.shape[1]
    HW = H * W
    dtype = x_nchw.dtype
    itemsize = jnp.dtype(dtype).itemsize

    hwp = _round_up(HW, 128)
    x = x_nchw.reshape(B, C, HW)
    if hwp != HW:
        x = jnp.pad(x, ((0, 0), (0, 0), (0, hwp - HW)))

    slice_bytes = C * hwp * itemsize
    nb = 1
    while nb < B and B % (nb * 2) == 0 and (nb * 2) * slice_bytes <= (4 << 20):
        nb *= 2

    vmem = int(min(max(4 * nb * slice_bytes + (4 << 20), 32 << 20), 100 << 20))

    out = pl.pallas_call(
        functools.partial(_se_kernel, inv_hw=1.0 / HW),
        out_shape=jax.ShapeDtypeStruct((B, C, hwp), dtype),
        grid=(B // nb,),
        in_specs=[
            pl.BlockSpec((nb, C, hwp), lambda b: (b, 0, 0)),
            pl.BlockSpec((C, Cr), lambda b: (0, 0)),
            pl.BlockSpec((1, Cr), lambda b: (0, 0)),
            pl.BlockSpec((Cr, C), lambda b: (0, 0)),
            pl.BlockSpec((1, C), lambda b: (0, 0)),
        ],
        out_specs=pl.BlockSpec((nb, C, hwp), lambda b: (b, 0, 0)),
        compiler_params=pltpu.CompilerParams(
            dimension_semantics=("arbitrary",),
            vmem_limit_bytes=vmem,
        ),
    )(
        x,
        w1.astype(jnp.float32),
        b1.reshape(1, Cr).astype(jnp.float32),
        w2.astype(jnp.float32),
        b2.reshape(1, C).astype(jnp.float32),
    )

    if hwp != HW:
        out = out[:, :, :HW]
    return out.reshape(B, C, H, W)


# speedup vs baseline: 1.0014x; 1.0014x over previous
import functools

import jax
import jax.numpy as jnp
from jax.experimental import pallas as pl
from jax.experimental.pallas import tpu as pltpu


def _round_up(n, m):
    return ((n + m - 1) // m) * m


def _se_kernel(xa_ref, xb_ref, w1_ref, b1_ref, w2_ref, b2_ref, o_ref,
               *, inv_hw):
    xa = xa_ref[...]
    xb = xb_ref[...]
    pooled = (jnp.sum(xa, axis=2, dtype=jnp.float32)
              + jnp.sum(xb, axis=2, dtype=jnp.float32)) * inv_hw
    h = jnp.dot(pooled, w1_ref[...],
                preferred_element_type=jnp.float32)
    h = jnp.maximum(h + b1_ref[...], 0.0)
    s = jnp.dot(h, w2_ref[...],
                preferred_element_type=jnp.float32)
    s = jax.nn.sigmoid(s + b2_ref[...])
    sb = s[:, :, None].astype(xa.dtype)
    half = xa.shape[2]
    o_ref[:, :, :half] = (xa * sb).astype(o_ref.dtype)
    o_ref[:, :, half:] = (xb * sb).astype(o_ref.dtype)


def kernel(x_nchw, w1, b1, w2, b2):
    B, C, H, W = x_nchw.shape
    Cr = w1.shape[1]
    HW = H * W
    dtype = x_nchw.dtype
    itemsize = jnp.dtype(dtype).itemsize

    hwp = _round_up(HW, 256)
    x = x_nchw.reshape(B, C, HW)
    if hwp != HW:
        x = jnp.pad(x, ((0, 0), (0, 0), (0, hwp - HW)))
    half = hwp // 2

    slice_bytes = C * hwp * itemsize
    nb = 1
    while nb < B and B % (nb * 2) == 0 and (nb * 2) * slice_bytes <= (4 << 20):
        nb *= 2

    vmem = int(min(max(4 * nb * slice_bytes + (4 << 20), 32 << 20), 100 << 20))

    out = pl.pallas_call(
        functools.partial(_se_kernel, inv_hw=1.0 / HW),
        out_shape=jax.ShapeDtypeStruct((B, C, hwp), dtype),
        grid=(B // nb,),
        in_specs=[
            pl.BlockSpec((nb, C, half), lambda b: (b, 0, 0)),
            pl.BlockSpec((nb, C, half), lambda b: (b, 0, 1)),
            pl.BlockSpec((C, Cr), lambda b: (0, 0)),
            pl.BlockSpec((1, Cr), lambda b: (0, 0)),
            pl.BlockSpec((Cr, C), lambda b: (0, 0)),
            pl.BlockSpec((1, C), lambda b: (0, 0)),
        ],
        out_specs=pl.BlockSpec((nb, C, hwp), lambda b: (b, 0, 0)),
        compiler_params=pltpu.CompilerParams(
            dimension_semantics=("arbitrary",),
            vmem_limit_bytes=vmem,
        ),
    )(
        x,
        x,
        w1.astype(jnp.float32),
        b1.reshape(1, Cr).astype(jnp.float32),
        w2.astype(jnp.float32),
        b2.reshape(1, C).astype(jnp.float32),
    )

    if hwp != HW:
        out = out[:, :, :HW]
    return out.reshape(B, C, H, W)


# speedup vs baseline: 1.0128x; 1.0114x over previous
import functools

import jax
import jax.numpy as jnp
from jax.experimental import pallas as pl
from jax.experimental.pallas import tpu as pltpu


def _round_up(n, m):
    return ((n + m - 1) // m) * m


def _se_kernel(xa_ref, xb_ref, w1_ref, b1_ref, w2_ref, b2_ref, o_ref,
               *, inv_hw):
    xa = xa_ref[...]
    xb = xb_ref[...]
    pooled = (jnp.sum(xa, axis=2, dtype=jnp.float32)
              + jnp.sum(xb, axis=2, dtype=jnp.float32)) * inv_hw
    h = jnp.dot(pooled, w1_ref[...],
                preferred_element_type=jnp.float32)
    h = jnp.maximum(h + b1_ref[...], 0.0)
    s = jnp.dot(h, w2_ref[...],
                preferred_element_type=jnp.float32)
    s = jax.nn.sigmoid(s + b2_ref[...])
    sb = s[:, :, None].astype(xa.dtype)
    half = xa.shape[2]
    o_ref[:, :, :half] = (xa * sb).astype(o_ref.dtype)
    o_ref[:, :, half:] = (xb * sb).astype(o_ref.dtype)


def kernel(x_nchw, w1, b1, w2, b2):
    B, C, H, W = x_nchw.shape
    Cr = w1.shape[1]
    HW = H * W
    dtype = x_nchw.dtype
    itemsize = jnp.dtype(dtype).itemsize

    hwp = _round_up(HW, 256)
    x = x_nchw.reshape(B, C, HW)
    if hwp != HW:
        x = jnp.pad(x, ((0, 0), (0, 0), (0, hwp - HW)))
    half = hwp // 2

    slice_bytes = C * hwp * itemsize
    nb = 1
    while nb < B and B % (nb * 2) == 0 and (nb * 2) * slice_bytes <= (8 << 20):
        nb *= 2

    vmem = int(min(max(4 * nb * slice_bytes + (4 << 20), 32 << 20), 100 << 20))

    out = pl.pallas_call(
        functools.partial(_se_kernel, inv_hw=1.0 / HW),
        out_shape=jax.ShapeDtypeStruct((B, C, hwp), dtype),
        grid=(B // nb,),
        in_specs=[
            pl.BlockSpec((nb, C, half), lambda b: (b, 0, 0)),
            pl.BlockSpec((nb, C, half), lambda b: (b, 0, 1)),
            pl.BlockSpec((C, Cr), lambda b: (0, 0)),
            pl.BlockSpec((1, Cr), lambda b: (0, 0)),
            pl.BlockSpec((Cr, C), lambda b: (0, 0)),
            pl.BlockSpec((1, C), lambda b: (0, 0)),
        ],
        out_specs=pl.BlockSpec((nb, C, hwp), lambda b: (b, 0, 0)),
        compiler_params=pltpu.CompilerParams(
            dimension_semantics=("arbitrary",),
            vmem_limit_bytes=vmem,
        ),
    )(
        x,
        x,
        w1.astype(jnp.float32),
        b1.reshape(1, Cr).astype(jnp.float32),
        w2.astype(jnp.float32),
        b2.reshape(1, C).astype(jnp.float32),
    )

    if hwp != HW:
        out = out[:, :, :HW]
    return out.reshape(B, C, H, W)
